# baseline (speedup 1.0000x reference)
import functools

import jax
import jax.numpy as jnp
from jax import lax
from jax.experimental import pallas as pl
from jax.experimental.pallas import tpu as pltpu
from jax.experimental.pallas import tpu_sc as plsc

_DIM = 2048
_N_EXPERTS = 64
_TOPK = 8
_BLOCK = 512
_TOKENS = 32768

_NW = 32
_ROWS_PER_W = _TOKENS // _NW
_IN_PER_W = _ROWS_PER_W * _N_EXPERTS
_OUT_PER_W = _ROWS_PER_W * _TOPK
_UNROLL = 4


def _probs_block_kernel(x_ref, wt_ref, p_ref):
    x = x_ref[...]
    wt = wt_ref[...]
    scores = lax.dot_general(
        x, wt, dimension_numbers=(((1,), (0,)), ((), ())),
        preferred_element_type=jnp.float32)
    m = jnp.max(scores, axis=-1, keepdims=True)
    e = jnp.exp(scores - m)
    p_ref[...] = e / jnp.sum(e, axis=-1, keepdims=True)


def _tc_probs(x, wt):
    grid = (_TOKENS // _BLOCK,)
    return pl.pallas_call(
        _probs_block_kernel,
        grid=grid,
        in_specs=[
            pl.BlockSpec((_BLOCK, _DIM), lambda i: (i, 0)),
            pl.BlockSpec((_DIM, _N_EXPERTS), lambda i: (0, 0)),
        ],
        out_specs=pl.BlockSpec((_BLOCK, _N_EXPERTS), lambda i: (i, 0)),
        out_shape=jax.ShapeDtypeStruct((_TOKENS, _N_EXPERTS), jnp.float32),
    )(x, wt)


def _merge_top16(ka, va, kb, vb):
    rk = lax.rev(kb, (0,))
    rv = lax.rev(vb, (0,))
    c = ka >= rk
    mk = jnp.where(c, ka, rk)
    mv = jnp.where(c, va, rv)
    return plsc.sort_key_val(mk, mv, descending=True)


def _sc_topk_body(probs_hbm, w_hbm, i_hbm, probs_v, w_v, i_v):
    wid = lax.axis_index("s") * 2 + lax.axis_index("c")
    pltpu.sync_copy(probs_hbm.at[pl.ds(wid * _IN_PER_W, _IN_PER_W)], probs_v)

    iota = lax.iota(jnp.int32, 16)
    m8 = iota < _TOPK
    idxs = [iota + 16 * j for j in range(4)]

    def do_row(r):
        base = r * _N_EXPERTS
        srt = [
            plsc.sort_key_val(probs_v[pl.ds(base + 16 * j, 16)], idxs[j],
                              descending=True)
            for j in range(4)
        ]
        k01, v01 = _merge_top16(*srt[0], *srt[1])
        k23, v23 = _merge_top16(*srt[2], *srt[3])
        kf, vf = _merge_top16(k01, v01, k23, v23)
        plsc.store_compressed(w_v.at[pl.ds(r * _TOPK, 16)], kf, mask=m8)
        plsc.store_compressed(i_v.at[pl.ds(r * _TOPK, 16)], vf, mask=m8)

    def body(i, carry):
        for u in range(_UNROLL):
            do_row(i * _UNROLL + u)
        return carry

    lax.fori_loop(0, _ROWS_PER_W // _UNROLL, body, 0, unroll=False)

    pltpu.sync_copy(w_v.at[pl.ds(0, _OUT_PER_W)],
                    w_hbm.at[pl.ds(wid * _OUT_PER_W, _OUT_PER_W)])
    pltpu.sync_copy(i_v.at[pl.ds(0, _OUT_PER_W)],
                    i_hbm.at[pl.ds(wid * _OUT_PER_W, _OUT_PER_W)])


def _sc_topk():
    return pl.kernel(
        _sc_topk_body,
        out_type=[
            jax.ShapeDtypeStruct((_TOKENS * _TOPK,), jnp.float32),
            jax.ShapeDtypeStruct((_TOKENS * _TOPK,), jnp.int32),
        ],
        mesh=plsc.VectorSubcoreMesh(core_axis_name="c", subcore_axis_name="s"),
        compiler_params=pltpu.CompilerParams(needs_layout_passes=False),
        scratch_types=[
            pltpu.VMEM((_IN_PER_W,), jnp.float32),
            pltpu.VMEM((_OUT_PER_W + 16,), jnp.float32),
            pltpu.VMEM((_OUT_PER_W + 16,), jnp.int32),
        ],
    )


def kernel(x, weight):
    wt = weight.T
    probs = _tc_probs(x, wt)
    w_flat, i_flat = _sc_topk()(probs.reshape(-1))
    return (w_flat.reshape(_TOKENS, _TOPK), i_flat.reshape(_TOKENS, _TOPK))
